# Initial kernel scaffold; baseline (speedup 1.0000x reference)
import functools

import jax
import jax.numpy as jnp
from jax import lax
from jax.experimental import pallas as pl
from jax.experimental.pallas import tpu as pltpu
from jax.experimental.pallas import tpu_sc as plsc

NW = 32
CHUNK = 80
ROWBLK = 500

_F32 = jnp.float32


def _sc_mesh():
    return plsc.VectorSubcoreMesh(core_axis_name="c", subcore_axis_name="s")


def _zero_acc_slice(zbuf, acc, s, zrows, rows_per_sub, f):
    zero16 = jnp.zeros((16,), _F32)

    def zb(i, carry):
        for j in range(f // 16):
            zbuf[i, pl.ds(j * 16, 16)] = zero16
        return carry

    lax.fori_loop(0, zrows, zb, 0)
    for k in range(rows_per_sub // zrows):
        pltpu.sync_copy(zbuf, acc.at[pl.ds(s * rows_per_sub + k * zrows, zrows)])


def _agg_sc(z, src_w, dst_w):
    n, f = z.shape
    nch = src_w.shape[1]
    rows_per_sub = n // 16
    zrows = 125

    @functools.partial(
        pl.kernel,
        mesh=_sc_mesh(),
        out_type=jax.ShapeDtypeStruct((2, n, f), _F32),
        scratch_types=[
            pltpu.VMEM((nch, CHUNK), jnp.int32),
            pltpu.VMEM((nch, CHUNK), jnp.int32),
            pltpu.VMEM((2, CHUNK, f), _F32),
            pltpu.VMEM((zrows, f), _F32),
            pltpu.VMEM_SHARED((n, f), _F32),
            pltpu.SemaphoreType.DMA,
        ],
    )
    def k(z_hbm, src_hbm, dst_hbm, out_hbm, sidx, didx, rows2, zbuf, acc, gsem):
        c = lax.axis_index("c")
        s = lax.axis_index("s")
        wid = s * 2 + c

        _zero_acc_slice(zbuf, acc, s, zrows, rows_per_sub, f)

        pltpu.sync_copy(src_hbm.at[wid], sidx)
        pltpu.sync_copy(dst_hbm.at[wid], didx)
        plsc.subcore_barrier()

        pltpu.async_copy(z_hbm.at[sidx.at[0]], rows2.at[0], gsem)

        def step(j, carry):
            p = lax.rem(j, 2)
            pltpu.make_async_copy(z_hbm.at[sidx.at[j]], rows2.at[p], gsem).wait()

            @pl.when(j + 1 < nch)
            def _():
                pltpu.async_copy(
                    z_hbm.at[sidx.at[j + 1]], rows2.at[lax.rem(j + 1, 2)], gsem)

            pltpu.sync_copy(rows2.at[p], acc.at[didx.at[j]], add=True)
            return carry

        lax.fori_loop(0, nch, step, 0)
        plsc.subcore_barrier()

        pltpu.sync_copy(
            acc.at[pl.ds(s * rows_per_sub, rows_per_sub)],
            out_hbm.at[c, pl.ds(s * rows_per_sub, rows_per_sub)])

    return k(z, src_w, dst_w)


def _deg_sc(dst_w, n):
    f = 16
    nch = dst_w.shape[1]
    rows_per_sub = n // 16
    zrows = 125

    @functools.partial(
        pl.kernel,
        mesh=_sc_mesh(),
        out_type=jax.ShapeDtypeStruct((2, n, f), _F32),
        scratch_types=[
            pltpu.VMEM((nch, CHUNK), jnp.int32),
            pltpu.VMEM((CHUNK, f), _F32),
            pltpu.VMEM((zrows, f), _F32),
            pltpu.VMEM_SHARED((n, f), _F32),
        ],
    )
    def k(dst_hbm, out_hbm, didx, onesb, zbuf, acc):
        c = lax.axis_index("c")
        s = lax.axis_index("s")
        wid = s * 2 + c

        _zero_acc_slice(zbuf, acc, s, zrows, rows_per_sub, f)

        lane = lax.iota(jnp.int32, 16)
        pat = jnp.where(lane == 0, 1.0, 0.0).astype(_F32)

        def ob(i, carry):
            onesb[i] = pat
            return carry

        lax.fori_loop(0, CHUNK, ob, 0)

        pltpu.sync_copy(dst_hbm.at[wid], didx)
        plsc.subcore_barrier()

        def step(j, carry):
            pltpu.sync_copy(onesb, acc.at[didx.at[j]], add=True)
            return carry

        lax.fori_loop(0, nch, step, 0)
        plsc.subcore_barrier()

        pltpu.sync_copy(
            acc.at[pl.ds(s * rows_per_sub, rows_per_sub)],
            out_hbm.at[c, pl.ds(s * rows_per_sub, rows_per_sub)])

    return k(dst_w)


def _tc1(x, w1, degp):
    n, d = x.shape
    h = w1.shape[1]

    def body(x_ref, w_ref, dg_ref, z_ref, dis_ref):
        deg = dg_ref[0, :, 0:1] + dg_ref[1, :, 0:1] + 1.0
        dis = lax.rsqrt(deg)
        dis_ref[...] = dis
        z_ref[...] = dis * jnp.dot(
            x_ref[...], w_ref[...], preferred_element_type=_F32)

    return pl.pallas_call(
        body,
        grid=(n // ROWBLK,),
        in_specs=[
            pl.BlockSpec((ROWBLK, d), lambda i: (i, 0)),
            pl.BlockSpec((d, h), lambda i: (0, 0)),
            pl.BlockSpec((2, ROWBLK, 16), lambda i: (0, i, 0)),
        ],
        out_specs=[
            pl.BlockSpec((ROWBLK, h), lambda i: (i, 0)),
            pl.BlockSpec((ROWBLK, 1), lambda i: (i, 0)),
        ],
        out_shape=[
            jax.ShapeDtypeStruct((n, h), _F32),
            jax.ShapeDtypeStruct((n, 1), _F32),
        ],
    )(x, w1, degp)


def _tc2(q, z1, dis, w2, b1):
    n, h1 = z1.shape
    h2 = w2.shape[1]

    def body(q_ref, z_ref, dis_ref, w_ref, b_ref, o_ref):
        dis = dis_ref[...]
        a = dis * (q_ref[0] + q_ref[1] + z_ref[...]) + b_ref[...]
        hr = jnp.maximum(a, 0.0)
        o_ref[...] = dis * jnp.dot(hr, w_ref[...], preferred_element_type=_F32)

    return pl.pallas_call(
        body,
        grid=(n // ROWBLK,),
        in_specs=[
            pl.BlockSpec((2, ROWBLK, h1), lambda i: (0, i, 0)),
            pl.BlockSpec((ROWBLK, h1), lambda i: (i, 0)),
            pl.BlockSpec((ROWBLK, 1), lambda i: (i, 0)),
            pl.BlockSpec((h1, h2), lambda i: (0, 0)),
            pl.BlockSpec((1, h1), lambda i: (0, 0)),
        ],
        out_specs=pl.BlockSpec((ROWBLK, h2), lambda i: (i, 0)),
        out_shape=jax.ShapeDtypeStruct((n, h2), _F32),
    )(q, z1, dis, w2, b1)


def _tc3(r, z2, dis, b2):
    n, h2 = z2.shape

    def body(r_ref, z_ref, dis_ref, b_ref, o_ref):
        dis = dis_ref[...]
        a = dis * (r_ref[0] + r_ref[1] + z_ref[...]) + b_ref[...]
        o_ref[...] = dis * jnp.maximum(a, 0.0)

    return pl.pallas_call(
        body,
        grid=(n // ROWBLK,),
        in_specs=[
            pl.BlockSpec((2, ROWBLK, h2), lambda i: (0, i, 0)),
            pl.BlockSpec((ROWBLK, h2), lambda i: (i, 0)),
            pl.BlockSpec((ROWBLK, 1), lambda i: (i, 0)),
            pl.BlockSpec((1, h2), lambda i: (0, 0)),
        ],
        out_specs=pl.BlockSpec((ROWBLK, h2), lambda i: (i, 0)),
        out_shape=jax.ShapeDtypeStruct((n, h2), _F32),
    )(r, z2, dis, b2)


def _tc4(sagg, z3, dis, w3, b3):
    n, h2 = z3.shape
    do = w3.shape[1]

    def body(s_ref, z_ref, dis_ref, w_ref, b_ref, o_ref):
        dis = dis_ref[...]
        a = dis * (s_ref[0] + s_ref[1] + z_ref[...])
        o = jnp.dot(a, w_ref[...], preferred_element_type=_F32) + b_ref[...]
        m = jnp.max(o, axis=1, keepdims=True)
        e = jnp.exp(o - m)
        lse = jnp.log(jnp.sum(e, axis=1, keepdims=True)) + m
        o_ref[...] = o - lse

    return pl.pallas_call(
        body,
        grid=(n // ROWBLK,),
        in_specs=[
            pl.BlockSpec((2, ROWBLK, h2), lambda i: (0, i, 0)),
            pl.BlockSpec((ROWBLK, h2), lambda i: (i, 0)),
            pl.BlockSpec((ROWBLK, 1), lambda i: (i, 0)),
            pl.BlockSpec((h2, do), lambda i: (0, 0)),
            pl.BlockSpec((1, do), lambda i: (0, 0)),
        ],
        out_specs=pl.BlockSpec((ROWBLK, do), lambda i: (i, 0)),
        out_shape=jax.ShapeDtypeStruct((n, do), _F32),
    )(sagg, z3, dis, w3, b3)


def kernel(x, edge_index, W1, b1, W2, b2, W3, b3):
    n = x.shape[0]
    e = edge_index.shape[1]
    epw = e // NW
    nch = epw // CHUNK
    assert epw * NW == e and nch * CHUNK == epw and n % (16 * 125) == 0

    src_w = edge_index[0].reshape(NW, nch, CHUNK)
    dst_w = edge_index[1].reshape(NW, nch, CHUNK)

    degp = _deg_sc(dst_w, n)
    z1, dis = _tc1(x, W1, degp)
    q = _agg_sc(z1, src_w, dst_w)
    z2 = _tc2(q, z1, dis, W2, b1.reshape(1, -1))
    r = _agg_sc(z2, src_w, dst_w)
    z3 = _tc3(r, z2, dis, b2.reshape(1, -1))
    s = _agg_sc(z3, src_w, dst_w)
    return _tc4(s, z3, dis, W3, b3.reshape(1, -1))


# speedup vs baseline: 26.2190x; 26.2190x over previous
import functools

import jax
import jax.numpy as jnp
from jax import lax
from jax.experimental import pallas as pl
from jax.experimental.pallas import tpu as pltpu
from jax.experimental.pallas import tpu_sc as plsc

NW = 32
CHUNK = 80
ROWBLK = 1000

_F32 = jnp.float32


def _sc_mesh():
    return plsc.VectorSubcoreMesh(core_axis_name="c", subcore_axis_name="s")


_RA = 624


def _zero_acc_slice(zbuf, acc, s, zrows, f):
    zero16 = jnp.zeros((16,), _F32)

    def zb(i, carry):
        for j in range(f // 16):
            zbuf[i, pl.ds(j * 16, 16)] = zero16
        return carry

    lax.fori_loop(0, zrows, zb, 0)
    for k in range(_RA // zrows):
        pltpu.sync_copy(zbuf, acc.at[pl.ds(s * _RA + k * zrows, zrows)])

    @pl.when(s == 15)
    def _():
        pltpu.sync_copy(zbuf.at[pl.ds(0, 16)], acc.at[pl.ds(16 * _RA, 16)])


def _copy_out_slice(acc, out_hbm, c, s):
    pltpu.sync_copy(acc.at[pl.ds(s * _RA, _RA)],
                    out_hbm.at[c, pl.ds(s * _RA, _RA)])

    @pl.when(s == 15)
    def _():
        pltpu.sync_copy(acc.at[pl.ds(16 * _RA, 16)],
                        out_hbm.at[c, pl.ds(16 * _RA, 16)])


def _agg_sc(z, src_w, dst_w):
    n, f = z.shape
    nch = src_w.shape[1]
    zrows = 312

    @functools.partial(
        pl.kernel,
        mesh=_sc_mesh(),
        out_type=jax.ShapeDtypeStruct((2, n, f), _F32),
        compiler_params=pltpu.CompilerParams(use_tc_tiling_on_sc=False),
        scratch_types=[
            pltpu.VMEM((nch, CHUNK), jnp.int32),
            pltpu.VMEM((nch, CHUNK), jnp.int32),
            pltpu.VMEM((2, CHUNK, f), _F32),
            pltpu.VMEM((zrows, f), _F32),
            pltpu.VMEM_SHARED((n, f), _F32),
            pltpu.SemaphoreType.DMA,
        ],
    )
    def k(z_hbm, src_hbm, dst_hbm, out_hbm, sidx, didx, rows2, zbuf, acc, gsem):
        c = lax.axis_index("c")
        s = lax.axis_index("s")
        wid = s * 2 + c

        _zero_acc_slice(zbuf, acc, s, zrows, f)

        pltpu.sync_copy(src_hbm.at[wid], sidx)
        pltpu.sync_copy(dst_hbm.at[wid], didx)
        plsc.subcore_barrier()

        pltpu.async_copy(z_hbm.at[sidx.at[0]], rows2.at[0], gsem)

        def step(j, carry):
            p = lax.rem(j, 2)
            pltpu.make_async_copy(z_hbm.at[sidx.at[j]], rows2.at[p], gsem).wait()

            @pl.when(j + 1 < nch)
            def _():
                pltpu.async_copy(
                    z_hbm.at[sidx.at[j + 1]], rows2.at[lax.rem(j + 1, 2)], gsem)

            pltpu.sync_copy(rows2.at[p], acc.at[didx.at[j]], add=True)
            return carry

        lax.fori_loop(0, nch, step, 0)
        plsc.subcore_barrier()

        _copy_out_slice(acc, out_hbm, c, s)

    return k(z, src_w, dst_w)


def _deg_sc(dst_w, n):
    f = 16
    nch = dst_w.shape[1]
    zrows = 312

    @functools.partial(
        pl.kernel,
        mesh=_sc_mesh(),
        out_type=jax.ShapeDtypeStruct((2, n, f), _F32),
        compiler_params=pltpu.CompilerParams(use_tc_tiling_on_sc=False),
        scratch_types=[
            pltpu.VMEM((nch, CHUNK), jnp.int32),
            pltpu.VMEM((CHUNK, f), _F32),
            pltpu.VMEM((zrows, f), _F32),
            pltpu.VMEM_SHARED((n, f), _F32),
        ],
    )
    def k(dst_hbm, out_hbm, didx, onesb, zbuf, acc):
        c = lax.axis_index("c")
        s = lax.axis_index("s")
        wid = s * 2 + c

        _zero_acc_slice(zbuf, acc, s, zrows, f)

        lane = lax.iota(jnp.int32, 16)
        pat = jnp.where(lane == 0, 1.0, 0.0).astype(_F32)

        def ob(i, carry):
            onesb[i] = pat
            return carry

        lax.fori_loop(0, CHUNK, ob, 0)

        pltpu.sync_copy(dst_hbm.at[wid], didx)
        plsc.subcore_barrier()

        def step(j, carry):
            pltpu.sync_copy(onesb, acc.at[didx.at[j]], add=True)
            return carry

        lax.fori_loop(0, nch, step, 0)
        plsc.subcore_barrier()

        _copy_out_slice(acc, out_hbm, c, s)

    return k(dst_w)


def _tc1(x, w1, degp):
    n, d = x.shape
    h = w1.shape[1]

    def body(x_ref, w_ref, dg_ref, z_ref, dis_ref):
        deg = dg_ref[0, :, 0:1] + dg_ref[1, :, 0:1] + 1.0
        dis = lax.rsqrt(deg)
        dis_ref[...] = dis
        z_ref[...] = dis * jnp.dot(
            x_ref[...], w_ref[...], preferred_element_type=_F32)

    return pl.pallas_call(
        body,
        grid=(n // ROWBLK,),
        in_specs=[
            pl.BlockSpec((ROWBLK, d), lambda i: (i, 0)),
            pl.BlockSpec((d, h), lambda i: (0, 0)),
            pl.BlockSpec((2, ROWBLK, 16), lambda i: (0, i, 0)),
        ],
        out_specs=[
            pl.BlockSpec((ROWBLK, h), lambda i: (i, 0)),
            pl.BlockSpec((ROWBLK, 1), lambda i: (i, 0)),
        ],
        out_shape=[
            jax.ShapeDtypeStruct((n, h), _F32),
            jax.ShapeDtypeStruct((n, 1), _F32),
        ],
    )(x, w1, degp)


def _tc2(q, z1, dis, w2, b1):
    n, h1 = z1.shape
    h2 = w2.shape[1]

    def body(q_ref, z_ref, dis_ref, w_ref, b_ref, o_ref):
        dis = dis_ref[...]
        a = dis * (q_ref[0] + q_ref[1] + z_ref[...]) + b_ref[...]
        hr = jnp.maximum(a, 0.0)
        o_ref[...] = dis * jnp.dot(hr, w_ref[...], preferred_element_type=_F32)

    return pl.pallas_call(
        body,
        grid=(n // ROWBLK,),
        in_specs=[
            pl.BlockSpec((2, ROWBLK, h1), lambda i: (0, i, 0)),
            pl.BlockSpec((ROWBLK, h1), lambda i: (i, 0)),
            pl.BlockSpec((ROWBLK, 1), lambda i: (i, 0)),
            pl.BlockSpec((h1, h2), lambda i: (0, 0)),
            pl.BlockSpec((1, h1), lambda i: (0, 0)),
        ],
        out_specs=pl.BlockSpec((ROWBLK, h2), lambda i: (i, 0)),
        out_shape=jax.ShapeDtypeStruct((n, h2), _F32),
    )(q, z1, dis, w2, b1)


def _tc3(r, z2, dis, b2):
    n, h2 = z2.shape

    def body(r_ref, z_ref, dis_ref, b_ref, o_ref):
        dis = dis_ref[...]
        a = dis * (r_ref[0] + r_ref[1] + z_ref[...]) + b_ref[...]
        o_ref[...] = dis * jnp.maximum(a, 0.0)

    return pl.pallas_call(
        body,
        grid=(n // ROWBLK,),
        in_specs=[
            pl.BlockSpec((2, ROWBLK, h2), lambda i: (0, i, 0)),
            pl.BlockSpec((ROWBLK, h2), lambda i: (i, 0)),
            pl.BlockSpec((ROWBLK, 1), lambda i: (i, 0)),
            pl.BlockSpec((1, h2), lambda i: (0, 0)),
        ],
        out_specs=pl.BlockSpec((ROWBLK, h2), lambda i: (i, 0)),
        out_shape=jax.ShapeDtypeStruct((n, h2), _F32),
    )(r, z2, dis, b2)


def _tc4(sagg, z3, dis, w3, b3):
    n, h2 = z3.shape
    do = w3.shape[1]

    def body(s_ref, z_ref, dis_ref, w_ref, b_ref, o_ref):
        dis = dis_ref[...]
        a = dis * (s_ref[0] + s_ref[1] + z_ref[...])
        o = jnp.dot(a, w_ref[...], preferred_element_type=_F32) + b_ref[...]
        m = jnp.max(o, axis=1, keepdims=True)
        e = jnp.exp(o - m)
        lse = jnp.log(jnp.sum(e, axis=1, keepdims=True)) + m
        o_ref[...] = o - lse

    return pl.pallas_call(
        body,
        grid=(n // ROWBLK,),
        in_specs=[
            pl.BlockSpec((2, ROWBLK, h2), lambda i: (0, i, 0)),
            pl.BlockSpec((ROWBLK, h2), lambda i: (i, 0)),
            pl.BlockSpec((ROWBLK, 1), lambda i: (i, 0)),
            pl.BlockSpec((h2, do), lambda i: (0, 0)),
            pl.BlockSpec((1, do), lambda i: (0, 0)),
        ],
        out_specs=pl.BlockSpec((ROWBLK, do), lambda i: (i, 0)),
        out_shape=jax.ShapeDtypeStruct((n, do), _F32),
    )(sagg, z3, dis, w3, b3)


def kernel(x, edge_index, W1, b1, W2, b2, W3, b3):
    n = x.shape[0]
    e = edge_index.shape[1]
    epw = e // NW
    nch = epw // CHUNK
    assert epw * NW == e and nch * CHUNK == epw and n == 16 * _RA + 16

    src_w = edge_index[0].reshape(NW, nch, CHUNK)
    dst_w = edge_index[1].reshape(NW, nch, CHUNK)

    degp = _deg_sc(dst_w, n)
    z1, dis = _tc1(x, W1, degp)
    q = _agg_sc(z1, src_w, dst_w)
    z2 = _tc2(q, z1, dis, W2, b1.reshape(1, -1))
    r = _agg_sc(z2, src_w, dst_w)
    z3 = _tc3(r, z2, dis, b2.reshape(1, -1))
    s = _agg_sc(z3, src_w, dst_w)
    return _tc4(s, z3, dis, W3, b3.reshape(1, -1))


# speedup vs baseline: 33.7505x; 1.2873x over previous
import functools

import jax
import jax.numpy as jnp
from jax import lax
from jax.experimental import pallas as pl
from jax.experimental.pallas import tpu as pltpu
from jax.experimental.pallas import tpu_sc as plsc

NW = 32
CHUNK = 128
NBUF = 4
DEGBUF = 8
PAD = 16
ROWBLK = 1000

_F32 = jnp.float32


def _sc_mesh():
    return plsc.VectorSubcoreMesh(core_axis_name="c", subcore_axis_name="s")


_RA = 624


def _zero_acc_slice(zbuf, acc, s, zrows, f):
    zero16 = jnp.zeros((16,), _F32)

    def zb(i, carry):
        for j in range(f // 16):
            zbuf[i, pl.ds(j * 16, 16)] = zero16
        return carry

    lax.fori_loop(0, zrows, zb, 0)
    for k in range(_RA // zrows):
        pltpu.sync_copy(zbuf, acc.at[pl.ds(s * _RA + k * zrows, zrows)])

    @pl.when(s == 15)
    def _():
        pltpu.sync_copy(zbuf.at[pl.ds(0, 16)], acc.at[pl.ds(16 * _RA, 16)])


def _copy_out_slice(acc, out_hbm, c, s):
    pltpu.sync_copy(acc.at[pl.ds(s * _RA, _RA)],
                    out_hbm.at[c, pl.ds(s * _RA, _RA)])

    @pl.when(s == 15)
    def _():
        pltpu.sync_copy(acc.at[pl.ds(16 * _RA, 16)],
                        out_hbm.at[c, pl.ds(16 * _RA, 16)])


def _agg_sc(z, src_w, dst_w):
    n, f = z.shape
    nch = src_w.shape[1]
    zrows = 312
    assert nch >= NBUF

    @functools.partial(
        pl.kernel,
        mesh=_sc_mesh(),
        out_type=jax.ShapeDtypeStruct((2, n, f), _F32),
        compiler_params=pltpu.CompilerParams(use_tc_tiling_on_sc=False),
        scratch_types=[
            pltpu.VMEM((nch, CHUNK), jnp.int32),
            pltpu.VMEM((nch, CHUNK), jnp.int32),
            pltpu.VMEM((NBUF, CHUNK, f), _F32),
            pltpu.VMEM((zrows, f), _F32),
            pltpu.VMEM_SHARED((n + PAD, f), _F32),
            pltpu.SemaphoreType.DMA((NBUF,)),
            pltpu.SemaphoreType.DMA((NBUF,)),
        ],
    )
    def k(z_hbm, src_hbm, dst_hbm, out_hbm, sidx, didx, rows, zbuf, acc,
          gsem, ssem):
        c = lax.axis_index("c")
        s = lax.axis_index("s")
        wid = s * 2 + c

        _zero_acc_slice(zbuf, acc, s, zrows, f)

        pltpu.sync_copy(src_hbm.at[wid], sidx)
        pltpu.sync_copy(dst_hbm.at[wid], didx)
        plsc.subcore_barrier()

        for b in range(NBUF):
            pltpu.async_copy(z_hbm.at[sidx.at[b]], rows.at[b], gsem.at[b])

        def step(j, carry):
            p = lax.rem(j, NBUF)
            pltpu.make_async_copy(
                z_hbm.at[sidx.at[j]], rows.at[p], gsem.at[p]).wait()
            pltpu.async_copy(
                rows.at[p], acc.at[didx.at[j]], ssem.at[p], add=True)

            @pl.when(j + NBUF < nch)
            def _():
                pltpu.make_async_copy(
                    rows.at[p], acc.at[didx.at[j]], ssem.at[p]).wait()
                pltpu.async_copy(
                    z_hbm.at[sidx.at[j + NBUF]], rows.at[p], gsem.at[p])

            return carry

        lax.fori_loop(0, nch, step, 0)
        for b in range(NBUF):
            pltpu.make_async_copy(
                rows.at[b], acc.at[didx.at[0]], ssem.at[b]).wait()
        plsc.subcore_barrier()

        _copy_out_slice(acc, out_hbm, c, s)

    return k(z, src_w, dst_w)


def _deg_sc(dst_w, n):
    f = 16
    nch = dst_w.shape[1]
    zrows = 312
    assert nch >= DEGBUF

    @functools.partial(
        pl.kernel,
        mesh=_sc_mesh(),
        out_type=jax.ShapeDtypeStruct((2, n, f), _F32),
        compiler_params=pltpu.CompilerParams(use_tc_tiling_on_sc=False),
        scratch_types=[
            pltpu.VMEM((nch, CHUNK), jnp.int32),
            pltpu.VMEM((CHUNK, f), _F32),
            pltpu.VMEM((zrows, f), _F32),
            pltpu.VMEM_SHARED((n + PAD, f), _F32),
            pltpu.SemaphoreType.DMA((DEGBUF,)),
        ],
    )
    def k(dst_hbm, out_hbm, didx, onesb, zbuf, acc, ssem):
        c = lax.axis_index("c")
        s = lax.axis_index("s")
        wid = s * 2 + c

        _zero_acc_slice(zbuf, acc, s, zrows, f)

        lane = lax.iota(jnp.int32, 16)
        pat = jnp.where(lane == 0, 1.0, 0.0).astype(_F32)

        def ob(i, carry):
            onesb[i] = pat
            return carry

        lax.fori_loop(0, CHUNK, ob, 0)

        pltpu.sync_copy(dst_hbm.at[wid], didx)
        plsc.subcore_barrier()

        for b in range(DEGBUF):
            pltpu.async_copy(onesb, acc.at[didx.at[b]], ssem.at[b], add=True)

        def step(j, carry):
            p = lax.rem(j, DEGBUF)
            pltpu.make_async_copy(onesb, acc.at[didx.at[j]], ssem.at[p]).wait()

            @pl.when(j + DEGBUF < nch)
            def _():
                pltpu.async_copy(
                    onesb, acc.at[didx.at[j + DEGBUF]], ssem.at[p], add=True)

            return carry

        lax.fori_loop(0, nch, step, 0)
        plsc.subcore_barrier()

        _copy_out_slice(acc, out_hbm, c, s)

    return k(dst_w)


def _tc1(x, w1, degp):
    n, d = x.shape
    h = w1.shape[1]

    def body(x_ref, w_ref, dg_ref, z_ref, dis_ref):
        deg = dg_ref[0, :, 0:1] + dg_ref[1, :, 0:1] + 1.0
        dis = lax.rsqrt(deg)
        dis_ref[...] = dis
        z_ref[...] = dis * jnp.dot(
            x_ref[...], w_ref[...], preferred_element_type=_F32)

    return pl.pallas_call(
        body,
        grid=(n // ROWBLK,),
        in_specs=[
            pl.BlockSpec((ROWBLK, d), lambda i: (i, 0)),
            pl.BlockSpec((d, h), lambda i: (0, 0)),
            pl.BlockSpec((2, ROWBLK, 16), lambda i: (0, i, 0)),
        ],
        out_specs=[
            pl.BlockSpec((ROWBLK, h), lambda i: (i, 0)),
            pl.BlockSpec((ROWBLK, 1), lambda i: (i, 0)),
        ],
        out_shape=[
            jax.ShapeDtypeStruct((n, h), _F32),
            jax.ShapeDtypeStruct((n, 1), _F32),
        ],
    )(x, w1, degp)


def _tc2(q, z1, dis, w2, b1):
    n, h1 = z1.shape
    h2 = w2.shape[1]

    def body(q_ref, z_ref, dis_ref, w_ref, b_ref, o_ref):
        dis = dis_ref[...]
        a = dis * (q_ref[0] + q_ref[1] + z_ref[...]) + b_ref[...]
        hr = jnp.maximum(a, 0.0)
        o_ref[...] = dis * jnp.dot(hr, w_ref[...], preferred_element_type=_F32)

    return pl.pallas_call(
        body,
        grid=(n // ROWBLK,),
        in_specs=[
            pl.BlockSpec((2, ROWBLK, h1), lambda i: (0, i, 0)),
            pl.BlockSpec((ROWBLK, h1), lambda i: (i, 0)),
            pl.BlockSpec((ROWBLK, 1), lambda i: (i, 0)),
            pl.BlockSpec((h1, h2), lambda i: (0, 0)),
            pl.BlockSpec((1, h1), lambda i: (0, 0)),
        ],
        out_specs=pl.BlockSpec((ROWBLK, h2), lambda i: (i, 0)),
        out_shape=jax.ShapeDtypeStruct((n, h2), _F32),
    )(q, z1, dis, w2, b1)


def _tc3(r, z2, dis, b2):
    n, h2 = z2.shape

    def body(r_ref, z_ref, dis_ref, b_ref, o_ref):
        dis = dis_ref[...]
        a = dis * (r_ref[0] + r_ref[1] + z_ref[...]) + b_ref[...]
        o_ref[...] = dis * jnp.maximum(a, 0.0)

    return pl.pallas_call(
        body,
        grid=(n // ROWBLK,),
        in_specs=[
            pl.BlockSpec((2, ROWBLK, h2), lambda i: (0, i, 0)),
            pl.BlockSpec((ROWBLK, h2), lambda i: (i, 0)),
            pl.BlockSpec((ROWBLK, 1), lambda i: (i, 0)),
            pl.BlockSpec((1, h2), lambda i: (0, 0)),
        ],
        out_specs=pl.BlockSpec((ROWBLK, h2), lambda i: (i, 0)),
        out_shape=jax.ShapeDtypeStruct((n, h2), _F32),
    )(r, z2, dis, b2)


def _tc4(sagg, z3, dis, w3, b3):
    n, h2 = z3.shape
    do = w3.shape[1]

    def body(s_ref, z_ref, dis_ref, w_ref, b_ref, o_ref):
        dis = dis_ref[...]
        a = dis * (s_ref[0] + s_ref[1] + z_ref[...])
        o = jnp.dot(a, w_ref[...], preferred_element_type=_F32) + b_ref[...]
        m = jnp.max(o, axis=1, keepdims=True)
        e = jnp.exp(o - m)
        lse = jnp.log(jnp.sum(e, axis=1, keepdims=True)) + m
        o_ref[...] = o - lse

    return pl.pallas_call(
        body,
        grid=(n // ROWBLK,),
        in_specs=[
            pl.BlockSpec((2, ROWBLK, h2), lambda i: (0, i, 0)),
            pl.BlockSpec((ROWBLK, h2), lambda i: (i, 0)),
            pl.BlockSpec((ROWBLK, 1), lambda i: (i, 0)),
            pl.BlockSpec((h2, do), lambda i: (0, 0)),
            pl.BlockSpec((1, do), lambda i: (0, 0)),
        ],
        out_specs=pl.BlockSpec((ROWBLK, do), lambda i: (i, 0)),
        out_shape=jax.ShapeDtypeStruct((n, do), _F32),
    )(sagg, z3, dis, w3, b3)


def kernel(x, edge_index, W1, b1, W2, b2, W3, b3):
    n = x.shape[0]
    e = edge_index.shape[1]
    assert n == 16 * _RA + 16

    blk = NW * CHUNK
    e_pad = -(-e // blk) * blk
    if e_pad != e:
        pad = jnp.broadcast_to(
            jnp.array([[0], [n]], dtype=edge_index.dtype), (2, e_pad - e))
        edge_index = jnp.concatenate([edge_index, pad], axis=1)
    nch = e_pad // blk

    src_w = edge_index[0].reshape(NW, nch, CHUNK)
    dst_w = edge_index[1].reshape(NW, nch, CHUNK)

    degp = _deg_sc(dst_w, n)
    z1, dis = _tc1(x, W1, degp)
    q = _agg_sc(z1, src_w, dst_w)
    z2 = _tc2(q, z1, dis, W2, b1.reshape(1, -1))
    r = _agg_sc(z2, src_w, dst_w)
    z3 = _tc3(r, z2, dis, b2.reshape(1, -1))
    s = _agg_sc(z3, src_w, dst_w)
    return _tc4(s, z3, dis, W3, b3.reshape(1, -1))
